# baseline (speedup 1.0000x reference)
import jax
import jax.numpy as jnp
from jax.experimental import pallas as pl

_B, _S, _VOCAB = 1024, 50, 1000
_SP, _VP = 56, 1024
_BB = 64
_H = _B // 2


def _body(idx_ref, o_ref):
    idx = idx_ref[...]
    cols = jax.lax.broadcasted_iota(jnp.int32, (_BB, _SP, _VP), 2)
    o_ref[...] = (cols == idx[:, :, None]).astype(jnp.float32)


_tc_onehot = pl.pallas_call(
    _body,
    out_shape=jax.ShapeDtypeStruct((_H, _SP, _VP), jnp.float32),
    grid=(_H // _BB,),
    in_specs=[pl.BlockSpec((_BB, _SP), lambda i: (i, 0))],
    out_specs=pl.BlockSpec((_BB, _SP, _VP), lambda i: (i, 0, 0)),
)


def kernel(inputs):
    idx = inputs.astype(jnp.int32)
    idx = jnp.pad(idx, ((0, 0), (0, _SP - _S)), constant_values=-1)
    a = _tc_onehot(idx[:_H])[:, :_S, :_VOCAB]
    b = _tc_onehot(idx[_H:])[:, :_S, :_VOCAB]
    return jnp.concatenate([a, b], axis=0)


# speedup vs baseline: 1.6226x; 1.6226x over previous
import jax
import jax.numpy as jnp
from jax.experimental import pallas as pl

_B, _S, _VOCAB = 1024, 50, 1000
_SP, _VP = 56, 1024
_BB = 64


def _body(idx_ref, o_ref):
    idx = idx_ref[...]
    cols = jax.lax.broadcasted_iota(jnp.int32, (_BB, _SP, _VP), 2)
    o_ref[...] = (cols == idx[:, :, None]).astype(jnp.float32)


_tc_onehot = pl.pallas_call(
    _body,
    out_shape=jax.ShapeDtypeStruct((_B, _SP, _VP), jnp.float32),
    grid=(_B // _BB,),
    in_specs=[pl.BlockSpec((_BB, _SP), lambda i: (i, 0))],
    out_specs=pl.BlockSpec((_BB, _SP, _VP), lambda i: (i, 0, 0)),
)


def kernel(inputs):
    idx = inputs.astype(jnp.int32)
    idx = jnp.pad(idx, ((0, 0), (0, _SP - _S)), constant_values=-1)
    return _tc_onehot(idx)[:, :_S, :_VOCAB]
